# Initial kernel scaffold; baseline (speedup 1.0000x reference)
import jax
import jax.numpy as jnp
from jax.experimental import pallas as pl


def kernel(x, Wg, W_gu, W_dn, Ws_gu, Ws_dn):
    raise NotImplementedError("write your pallas kernel here")


# speedup vs baseline: 3.7820x; 3.7820x over previous
import functools

import jax
import jax.numpy as jnp
from jax.experimental import pallas as pl
from jax.experimental.pallas import tpu as pltpu

_H = 1024
_I = 512
_E = 64
_NG = 8
_GS = _E // _NG
_TOPKG = 4
_TOPK = 8
_NSH = 2
_NT = 512

_NEG = float("-inf")


def _gate_kernel(x_ref, wgt_ref, combine_ref, idx_ref, w_ref):
    x = x_ref[...]
    wgt = wgt_ref[...]
    logits = jnp.dot(x, wgt, preferred_element_type=jnp.float32)

    lane = jax.lax.broadcasted_iota(jnp.int32, (_NT, _E), 1)
    lanef = lane.astype(jnp.float32)
    group = lane // _GS

    def seg_max(v):
        gm = jnp.full((_NT, _E), _NEG, jnp.float32)
        for g in range(_NG):
            in_g = group == g
            mg = jnp.max(jnp.where(in_g, v, _NEG), axis=1, keepdims=True)
            gm = jnp.where(in_g, mg, gm)
        return gm

    def seg_min(v):
        gm = jnp.full((_NT, _E), jnp.float32(1e9), jnp.float32)
        for g in range(_NG):
            in_g = group == g
            mg = jnp.min(jnp.where(in_g, v, 1e9), axis=1, keepdims=True)
            gm = jnp.where(in_g, mg, gm)
        return gm

    active = jnp.ones((_NT, _E), jnp.bool_)
    sel4 = jnp.zeros((_NT, _E), jnp.bool_)
    for _ in range(_TOPKG):
        v = jnp.where(active, logits, _NEG)
        gm = seg_max(v)
        ismax = (v == gm) & active
        fm = seg_min(jnp.where(ismax, lanef, 1e9))
        s = ismax & (lanef == fm)
        sel4 = sel4 | s
        active = active & (~s)

    active = sel4
    sel8 = jnp.zeros((_NT, _E), jnp.bool_)
    for _ in range(_TOPK):
        v = jnp.where(active, logits, _NEG)
        m = jnp.max(v, axis=1, keepdims=True)
        ismax = (v == m) & active
        fm = jnp.min(jnp.where(ismax, lanef, 1e9), axis=1, keepdims=True)
        s = ismax & (lanef == fm)
        sel8 = sel8 | s
        active = active & (~s)

    wsel = jnp.where(sel8, logits, 0.0)
    denom = jnp.sum(wsel, axis=1, keepdims=True) + 1e-20
    combine = wsel / denom
    combine_ref[...] = combine

    sel_f = sel8.astype(jnp.float32)
    row = jax.lax.broadcasted_iota(jnp.int32, (_E, _E), 0)
    col = jax.lax.broadcasted_iota(jnp.int32, (_E, _E), 1)
    strict_lower = (row < col).astype(jnp.float32)
    rank = jnp.dot(sel_f, strict_lower, preferred_element_type=jnp.float32)
    idx_cols = []
    w_cols = []
    for k in range(_TOPK):
        mk = sel8 & (rank == k)
        idx_cols.append(jnp.sum(jnp.where(mk, lanef, 0.0), axis=1, keepdims=True))
        w_cols.append(jnp.sum(jnp.where(mk, combine, 0.0), axis=1, keepdims=True))
    idx_ref[...] = jnp.concatenate(idx_cols, axis=1).astype(jnp.int32)
    w_ref[...] = jnp.concatenate(w_cols, axis=1)


def _gate(x, WgT):
    return pl.pallas_call(
        _gate_kernel,
        out_shape=(
            jax.ShapeDtypeStruct((_NT, _E), jnp.float32),
            jax.ShapeDtypeStruct((_NT, _TOPK), jnp.int32),
            jax.ShapeDtypeStruct((_NT, _TOPK), jnp.float32),
        ),
    )(x, WgT)


def _silu(v):
    return v / (1.0 + jnp.exp(-v))


def _dense_kernel(combine_ref, x_ref, wgu_ref, wdn_ref, wsgu_ref, wsdn_ref,
                  out_ref):
    pid = pl.program_id(0)

    @pl.when(pid == 0)
    def _():
        out_ref[...] = jnp.zeros_like(out_ref)

    def mlp(gu, dn):
        h = jnp.dot(x_ref[...], gu, preferred_element_type=jnp.float32)
        act = _silu(h[:, :_I]) * h[:, _I:]
        return jnp.dot(act, dn, preferred_element_type=jnp.float32)

    @pl.when(pid < _E)
    def _():
        onehot = (jax.lax.broadcasted_iota(jnp.int32, (_E, 1), 0) == pid)
        c = jnp.dot(combine_ref[...], onehot.astype(jnp.float32),
                    preferred_element_type=jnp.float32)
        out_ref[...] += c * mlp(wgu_ref[0], wdn_ref[0])

    @pl.when(pid >= _E)
    def _():
        out_ref[...] += mlp(wsgu_ref[0], wsdn_ref[0])


def _moe_dense(combine, x, W_gu, W_dn, Ws_gu, Ws_dn):
    grid = (_E + _NSH,)
    return pl.pallas_call(
        _dense_kernel,
        grid=grid,
        in_specs=[
            pl.BlockSpec((_NT, _E), lambda e: (0, 0)),
            pl.BlockSpec((_NT, _H), lambda e: (0, 0)),
            pl.BlockSpec((1, _H, 2 * _I), lambda e: (jnp.minimum(e, _E - 1), 0, 0)),
            pl.BlockSpec((1, _I, _H), lambda e: (jnp.minimum(e, _E - 1), 0, 0)),
            pl.BlockSpec((1, _H, 2 * _I), lambda e: (jnp.clip(e - _E, 0, _NSH - 1), 0, 0)),
            pl.BlockSpec((1, _I, _H), lambda e: (jnp.clip(e - _E, 0, _NSH - 1), 0, 0)),
        ],
        out_specs=pl.BlockSpec((_NT, _H), lambda e: (0, 0)),
        out_shape=jax.ShapeDtypeStruct((_NT, _H), jnp.float32),
        compiler_params=pltpu.CompilerParams(
            dimension_semantics=("arbitrary",),
        ),
    )(combine, x, W_gu, W_dn, Ws_gu, Ws_dn)


@jax.jit
def kernel(x, Wg, W_gu, W_dn, Ws_gu, Ws_dn):
    combine, _, _ = _gate(x, Wg.T)
    return _moe_dense(combine, x, W_gu, W_dn, Ws_gu, Ws_dn)
